# baseline (speedup 1.0000x reference)
import functools

import jax
import jax.numpy as jnp
from jax import lax
from jax.experimental import pallas as pl
from jax.experimental.pallas import tpu as pltpu
from jax.experimental.pallas import tpu_sc as plsc

_K = 8
_H = 16
_T = 2048
_S = 2048
_SAMPLED_HEAD = 12
_L = 16
_NCHUNK = _S // _L
_BR = 512
_NB = (_H * _T) // _BR

_MESH = plsc.VectorSubcoreMesh(
    core_axis_name="c", subcore_axis_name="s", num_cores=1, num_subcores=16
)


def _sc_slice_kernel(aw_hbm, newx_hbm, row_v, r12_v, pub_v, all_v, shared,
                     sem_a, sem_b):
    sid = lax.axis_index("s")

    own_cp = pltpu.make_async_copy(aw_hbm.at[sid, _T - 1], row_v, sem_a)
    own_cp.start()
    r12_cp = pltpu.make_async_copy(
        aw_hbm.at[_SAMPLED_HEAD, _T - 1], r12_v, sem_b
    )
    r12_cp.start()
    own_cp.wait()

    lanes = lax.iota(jnp.int32, _L)

    def body(j, carry):
        best_val, best_idx = carry
        v = row_v[pl.ds(j * _L, _L)]
        cur_idx = j * _L + lanes
        take = v > best_val
        return (
            jnp.where(take, v, best_val),
            jnp.where(take, cur_idx, best_idx),
        )

    best_val, best_idx = lax.fori_loop(
        0,
        _NCHUNK,
        body,
        (jnp.full((_L,), -jnp.inf, jnp.float32), jnp.zeros((_L,), jnp.int32)),
    )
    m = jnp.max(best_val)
    arg = jnp.min(jnp.where(best_val == m, best_idx, _S))

    pub_v[...] = jnp.full((_L,), arg, jnp.int32)
    pltpu.sync_copy(pub_v, shared.at[sid])
    plsc.subcore_barrier()
    pltpu.sync_copy(shared, all_v)

    argvec = plsc.load_gather(all_v, [lanes, lanes])
    counts = jnp.zeros((_L,), jnp.int32)
    for j in range(_H):
        counts = counts + jnp.where(argvec == all_v[j], 1, 0)
    cond = jnp.max(counts) <= _K

    r12_cp.wait()

    @pl.when(cond)
    def _():
        pltpu.sync_copy(r12_v, newx_hbm.at[sid])

    @pl.when(jnp.logical_not(cond))
    def _():
        pltpu.sync_copy(row_v, newx_hbm.at[sid])


_sc_slice = functools.partial(
    pl.kernel,
    out_type=jax.ShapeDtypeStruct((_H, _S), jnp.float32),
    mesh=_MESH,
    compiler_params=pltpu.CompilerParams(needs_layout_passes=False),
    scratch_types=[
        pltpu.VMEM((_S,), jnp.float32),
        pltpu.VMEM((_S,), jnp.float32),
        pltpu.VMEM((_L,), jnp.int32),
        pltpu.VMEM((_H, _L), jnp.int32),
        pltpu.VMEM_SHARED((_H, _L), jnp.int32),
        pltpu.SemaphoreType.DMA,
        pltpu.SemaphoreType.DMA,
    ],
)(_sc_slice_kernel)


def _copy_kernel(flat_ref, out_ref):
    out_ref[...] = flat_ref[...]


_SLAB = 8


def _patch_kernel(slab_ref, newx_ref, out_ref):
    slab = slab_ref[...]
    t_idx = jax.lax.broadcasted_iota(jnp.int32, slab.shape, 1)
    out_ref[...] = jnp.where(
        t_idx == _SLAB - 1, newx_ref[...][:, None, :], slab
    )


def kernel(attention_weight):
    aw3 = attention_weight.reshape(_H, _T, _S)
    flat = attention_weight.reshape(_H * _T, _S)
    newx = _sc_slice(aw3)
    tmp = pl.pallas_call(
        _copy_kernel,
        grid=(_NB,),
        in_specs=[pl.BlockSpec((_BR, _S), lambda i: (i, 0))],
        out_specs=pl.BlockSpec((_BR, _S), lambda i: (i, 0)),
        out_shape=jax.ShapeDtypeStruct((_H * _T, _S), jnp.float32),
        compiler_params=pltpu.CompilerParams(
            dimension_semantics=("arbitrary",),
        ),
    )(flat)
    blk = (_H, _SLAB, _S)
    last_blk = (_T - _SLAB) // _SLAB
    out = pl.pallas_call(
        _patch_kernel,
        grid=(1,),
        in_specs=[
            pl.BlockSpec(blk, lambda i: (0, last_blk, 0)),
            pl.BlockSpec((_H, _S), lambda i: (0, 0)),
        ],
        out_specs=pl.BlockSpec(blk, lambda i: (0, last_blk, 0)),
        out_shape=jax.ShapeDtypeStruct((_H, _T, _S), jnp.float32),
        input_output_aliases={0: 0},
    )(tmp.reshape(_H, _T, _S), newx)
    return out.reshape(1, _H, _T, _S)


# speedup vs baseline: 1.0188x; 1.0188x over previous
import functools

import jax
import jax.numpy as jnp
from jax import lax
from jax.experimental import pallas as pl
from jax.experimental.pallas import tpu as pltpu
from jax.experimental.pallas import tpu_sc as plsc

_K = 8
_H = 16
_T = 2048
_S = 2048
_SAMPLED_HEAD = 12
_L = 16
_NCHUNK = _S // _L
_BR = 1024
_NB = (_H * _T) // _BR

_MESH = plsc.VectorSubcoreMesh(
    core_axis_name="c", subcore_axis_name="s", num_cores=1, num_subcores=16
)


def _sc_slice_kernel(aw_hbm, newx_hbm, row_v, r12_v, pub_v, all_v, shared,
                     sem_a, sem_b):
    sid = lax.axis_index("s")

    own_cp = pltpu.make_async_copy(aw_hbm.at[sid, _T - 1], row_v, sem_a)
    own_cp.start()
    r12_cp = pltpu.make_async_copy(
        aw_hbm.at[_SAMPLED_HEAD, _T - 1], r12_v, sem_b
    )
    r12_cp.start()
    own_cp.wait()

    lanes = lax.iota(jnp.int32, _L)

    def body(j, carry):
        best_val, best_idx = carry
        v = row_v[pl.ds(j * _L, _L)]
        cur_idx = j * _L + lanes
        take = v > best_val
        return (
            jnp.where(take, v, best_val),
            jnp.where(take, cur_idx, best_idx),
        )

    best_val, best_idx = lax.fori_loop(
        0,
        _NCHUNK,
        body,
        (jnp.full((_L,), -jnp.inf, jnp.float32), jnp.zeros((_L,), jnp.int32)),
    )
    m = jnp.max(best_val)
    arg = jnp.min(jnp.where(best_val == m, best_idx, _S))

    pub_v[...] = jnp.full((_L,), arg, jnp.int32)
    pltpu.sync_copy(pub_v, shared.at[sid])
    plsc.subcore_barrier()
    pltpu.sync_copy(shared, all_v)

    argvec = plsc.load_gather(all_v, [lanes, lanes])
    counts = jnp.zeros((_L,), jnp.int32)
    for j in range(_H):
        counts = counts + jnp.where(argvec == all_v[j], 1, 0)
    cond = jnp.max(counts) <= _K

    r12_cp.wait()

    @pl.when(cond)
    def _():
        pltpu.sync_copy(r12_v, newx_hbm.at[sid])

    @pl.when(jnp.logical_not(cond))
    def _():
        pltpu.sync_copy(row_v, newx_hbm.at[sid])


_sc_slice = functools.partial(
    pl.kernel,
    out_type=jax.ShapeDtypeStruct((_H, _S), jnp.float32),
    mesh=_MESH,
    compiler_params=pltpu.CompilerParams(needs_layout_passes=False),
    scratch_types=[
        pltpu.VMEM((_S,), jnp.float32),
        pltpu.VMEM((_S,), jnp.float32),
        pltpu.VMEM((_L,), jnp.int32),
        pltpu.VMEM((_H, _L), jnp.int32),
        pltpu.VMEM_SHARED((_H, _L), jnp.int32),
        pltpu.SemaphoreType.DMA,
        pltpu.SemaphoreType.DMA,
    ],
)(_sc_slice_kernel)


def _copy_kernel(flat_ref, out_ref):
    out_ref[...] = flat_ref[...]


_SLAB = 8


def _patch_kernel(slab_ref, newx_ref, out_ref):
    slab = slab_ref[...]
    t_idx = jax.lax.broadcasted_iota(jnp.int32, slab.shape, 1)
    out_ref[...] = jnp.where(
        t_idx == _SLAB - 1, newx_ref[...][:, None, :], slab
    )


def kernel(attention_weight):
    aw3 = attention_weight.reshape(_H, _T, _S)
    flat = attention_weight.reshape(_H * _T, _S)
    newx = _sc_slice(aw3)
    tmp = pl.pallas_call(
        _copy_kernel,
        grid=(_NB,),
        in_specs=[pl.BlockSpec((_BR, _S), lambda i: (i, 0))],
        out_specs=pl.BlockSpec((_BR, _S), lambda i: (i, 0)),
        out_shape=jax.ShapeDtypeStruct((_H * _T, _S), jnp.float32),
        compiler_params=pltpu.CompilerParams(
            dimension_semantics=("arbitrary",),
        ),
    )(flat)
    blk = (_H, _SLAB, _S)
    last_blk = (_T - _SLAB) // _SLAB
    out = pl.pallas_call(
        _patch_kernel,
        grid=(1,),
        in_specs=[
            pl.BlockSpec(blk, lambda i: (0, last_blk, 0)),
            pl.BlockSpec((_H, _S), lambda i: (0, 0)),
        ],
        out_specs=pl.BlockSpec(blk, lambda i: (0, last_blk, 0)),
        out_shape=jax.ShapeDtypeStruct((_H, _T, _S), jnp.float32),
        input_output_aliases={0: 0},
    )(tmp.reshape(_H, _T, _S), newx)
    return out.reshape(1, _H, _T, _S)
